# baseline (speedup 1.0000x reference)
import functools

import jax
import jax.numpy as jnp
from jax import lax
from jax.experimental import pallas as pl
from jax.experimental.pallas import tpu as pltpu
from jax.experimental.pallas import tpu_sc as plsc

CODEBOOK = 8192
DIM = 768
COMMIT = 0.25
LANES = 16

R_SC = 4096
R_TC = CODEBOOK
NB = 16
BKT = R_TC // NB
NBUF = 4

NTILES = 32
RPT = R_SC // NTILES
CH = 8
NCH = RPT // CH
NSUP = NCH // 2
NZC = DIM // LANES


def _dots_body(z_ref, w_hbm, tmax_ref, tidx_ref, trow_ref, bufs, sems):
    zb = z_ref[...]

    def start(c):
        slot = c % NBUF
        pltpu.make_async_copy(
            w_hbm.at[pl.ds(c * BKT, BKT), :], bufs.at[slot], sems.at[slot]
        ).start()

    for c in range(min(NBUF, NB)):
        start(c)
    best_m = jnp.float32(-jnp.inf)
    best_i = jnp.int32(0)
    best_row = jnp.zeros((1, DIM), jnp.float32)
    for c in range(NB):
        slot = c % NBUF
        pltpu.make_async_copy(
            w_hbm.at[pl.ds(c * BKT, BKT), :], bufs.at[slot], sems.at[slot]
        ).wait()
        if c + NBUF < NB:
            start(c + NBUF)
        wb = bufs[slot]
        dots = lax.dot_general(wb, zb, (((1,), (0,)), ((), ())),
                               preferred_element_type=jnp.float32)
        m = jnp.max(dots)
        iota = lax.broadcasted_iota(jnp.int32, (BKT, 1), 0)
        cand = jnp.where(dots == m, iota, jnp.int32(BKT))
        a = jnp.min(cand)
        row = bufs[slot, pl.ds(a, 1), :]
        better = m > best_m
        best_row = jnp.where(better, row, best_row)
        best_i = jnp.where(better, a + c * BKT, best_i)
        best_m = jnp.where(better, m, best_m)
    tmax_ref[0] = best_m
    tidx_ref[0] = best_i
    trow_ref[...] = best_row


_dots_call = pl.pallas_call(
    _dots_body,
    in_specs=[
        pl.BlockSpec(memory_space=pltpu.VMEM),
        pl.BlockSpec(memory_space=pl.ANY),
    ],
    out_specs=[
        pl.BlockSpec(memory_space=pltpu.SMEM),
        pl.BlockSpec(memory_space=pltpu.SMEM),
        pl.BlockSpec(memory_space=pltpu.VMEM),
    ],
    out_shape=[
        jax.ShapeDtypeStruct((1,), jnp.float32),
        jax.ShapeDtypeStruct((1,), jnp.int32),
        jax.ShapeDtypeStruct((1, DIM), jnp.float32),
    ],
    scratch_shapes=[
        pltpu.VMEM((NBUF, BKT, DIM), jnp.float32),
        pltpu.SemaphoreType.DMA((NBUF,)),
    ],
)


_sc_mesh = plsc.VectorSubcoreMesh(core_axis_name="c", subcore_axis_name="s")


@functools.partial(
    pl.kernel,
    mesh=_sc_mesh,
    compiler_params=pltpu.CompilerParams(needs_layout_passes=False),
    out_type=(
        jax.ShapeDtypeStruct((2, LANES), jnp.float32),
        jax.ShapeDtypeStruct((2, LANES), jnp.int32),
        jax.ShapeDtypeStruct((2, DIM), jnp.float32),
    ),
    scratch_types=[
        pltpu.VMEM((DIM,), jnp.float32),
        pltpu.VMEM((CH, DIM), jnp.float32),
        pltpu.VMEM((CH, DIM), jnp.float32),
        pltpu.VMEM((LANES,), jnp.float32),
        pltpu.VMEM((LANES,), jnp.float32),
        pltpu.VMEM((LANES,), jnp.int32),
        pltpu.VMEM((LANES, LANES), jnp.float32),
        pltpu.VMEM((LANES, LANES), jnp.int32),
        pltpu.VMEM((LANES,), jnp.int32),
        pltpu.VMEM((LANES, DIM), jnp.float32),
        pltpu.VMEM_SHARED((LANES, LANES), jnp.float32),
        pltpu.VMEM_SHARED((LANES, LANES), jnp.int32),
        pltpu.SemaphoreType.DMA,
        pltpu.SemaphoreType.DMA,
        pltpu.SemaphoreType.DMA,
    ],
)
def _sc_shard(w_hbm, z_hbm, scmax_hbm, scidx_hbm, scrows_hbm,
              z_v, buf0, buf1, acc_v, best_v, bidx_v, allmax_v, allidx_v,
              idxsel_v, rows_v, shared_max, shared_idx, sem0, sem1, semg):
    cid = lax.axis_index("c")
    sid = lax.axis_index("s")
    row0 = R_TC + (cid * LANES + sid) * RPT
    lane = lax.broadcasted_iota(jnp.int32, (LANES,), 0)

    pltpu.sync_copy(z_hbm, z_v)

    def start_chunk(k, buf, sem):
        pltpu.make_async_copy(
            w_hbm.at[pl.ds(row0 + k * CH, CH), :], buf, sem
        ).start()

    start_chunk(0, buf0, sem0)
    start_chunk(1, buf1, sem1)

    def do_chunk(k, buf, sem, nxt_buf_sem, carry):
        best, bidx = carry
        pltpu.make_async_copy(
            w_hbm.at[pl.ds(row0 + k * CH, CH), :], buf, sem
        ).wait()
        accs = [jnp.zeros((LANES,), jnp.float32) for _ in range(CH)]
        for j in range(NZC):
            sl = pl.ds(j * LANES, LANES)
            zj = z_v[sl]
            for r in range(CH):
                accs[r] = accs[r] + buf[r, sl] * zj
        @pl.when(k + 2 < NCH)
        def _():
            pltpu.make_async_copy(
                w_hbm.at[pl.ds(row0 + (k + 2) * CH, CH), :], buf, sem
            ).start()
        for r in range(CH):
            acc_v[...] = accs[r]
            for stp in (1, 2, 4, 8):
                acc_v[...] = acc_v[...] + plsc.load_gather(acc_v,
                                                           [lane ^ stp])
            dot = acc_v[...]
            better = dot > best
            ridx = row0 + k * CH + r
            best = jnp.where(better, dot, best)
            bidx = jnp.where(better,
                             jnp.full((LANES,), ridx, jnp.int32), bidx)
        return best, bidx

    def super_step(i, carry):
        k = i * 2
        carry = do_chunk(k, buf0, sem0, None, carry)
        carry = do_chunk(k + 1, buf1, sem1, None, carry)
        return carry

    init = (jnp.full((LANES,), -jnp.inf, jnp.float32),
            jnp.zeros((LANES,), jnp.int32))
    best, bidx = lax.fori_loop(0, NSUP, super_step, init)

    best_v[...] = best
    bidx_v[...] = bidx
    pltpu.sync_copy(best_v, shared_max.at[sid])
    pltpu.sync_copy(bidx_v, shared_idx.at[sid])
    plsc.subcore_barrier()

    @pl.when(sid == 0)
    def _():
        pltpu.sync_copy(shared_max, allmax_v)
        pltpu.sync_copy(shared_idx, allidx_v)
        gbest = allmax_v[0, :]
        gidx = allidx_v[0, :]
        for s in range(1, LANES):
            v = allmax_v[s, :]
            ii = allidx_v[s, :]
            better = v > gbest
            gbest = jnp.where(better, v, gbest)
            gidx = jnp.where(better, ii, gidx)
        best_v[...] = gbest
        idxsel_v[...] = gidx
        pltpu.async_copy(w_hbm.at[idxsel_v], rows_v, semg).wait()
        pltpu.sync_copy(best_v, scmax_hbm.at[cid])
        pltpu.sync_copy(idxsel_v, scidx_hbm.at[cid])
        pltpu.sync_copy(rows_v.at[0], scrows_hbm.at[cid])


def _merge_body(tmax_ref, tidx_ref, trow_ref, scmax_ref, scidx_ref,
                scrows_ref, z_ref, qst_ref, idx_ref, loss_ref):
    m0 = tmax_ref[0]
    i0 = tidx_ref[0]
    m1 = scmax_ref[0, 0]
    i1 = scidx_ref[0, 0]
    m2 = scmax_ref[1, 0]
    i2 = scidx_ref[1, 0]
    b1 = m1 > m0
    mA = jnp.where(b1, m1, m0)
    iA = jnp.where(b1, i1, i0)
    selA = jnp.where(b1, jnp.int32(1), jnp.int32(0))
    b2 = m2 > mA
    iW = jnp.where(b2, i2, iA)
    sel = jnp.where(b2, jnp.int32(2), selA)
    row0 = trow_ref[0, :]
    row1 = scrows_ref[0, :]
    row2 = scrows_ref[1, :]
    q = jnp.where(sel == 0, row0, jnp.where(sel == 1, row1, row2))
    zb = z_ref[0, :]
    d = zb - q
    qst_ref[0, :] = zb - d
    loss = jnp.float32(COMMIT) * (jnp.sum(d * d) / jnp.float32(DIM))
    idx_ref[0] = iW
    loss_ref[0] = loss


_merge_call = pl.pallas_call(
    _merge_body,
    in_specs=[
        pl.BlockSpec(memory_space=pltpu.SMEM),
        pl.BlockSpec(memory_space=pltpu.SMEM),
        pl.BlockSpec(memory_space=pltpu.VMEM),
        pl.BlockSpec(memory_space=pltpu.SMEM),
        pl.BlockSpec(memory_space=pltpu.SMEM),
        pl.BlockSpec(memory_space=pltpu.VMEM),
        pl.BlockSpec(memory_space=pltpu.VMEM),
    ],
    out_specs=[
        pl.BlockSpec(memory_space=pltpu.VMEM),
        pl.BlockSpec(memory_space=pltpu.SMEM),
        pl.BlockSpec(memory_space=pltpu.SMEM),
    ],
    out_shape=[
        jax.ShapeDtypeStruct((1, DIM), jnp.float32),
        jax.ShapeDtypeStruct((1,), jnp.int32),
        jax.ShapeDtypeStruct((1,), jnp.float32),
    ],
)


def kernel(z, W):
    scmax = jnp.full((2, LANES), -jnp.inf, jnp.float32)
    scidx = jnp.zeros((2, LANES), jnp.int32)
    scrows = jnp.zeros((2, DIM), jnp.float32)
    tmax, tidx, trow = _dots_call(z[:, None], W)
    qst2, idxv, lossv = _merge_call(tmax, tidx, trow, scmax, scidx,
                                    scrows, z[None, :])
    return qst2[0], idxv[0], lossv[0]
